# baseline (speedup 1.0000x reference)
import functools

import jax
import jax.numpy as jnp
from jax import lax
from jax.experimental import pallas as pl
from jax.experimental.pallas import tpu as pltpu
from jax.experimental.pallas import tpu_sc as plsc

_RES = 128
_N_RAYS = 65536
_N_SAMPLES = 128
_OCC_THRE = 0.5

_NC = 2
_NS = 16
_NW = _NC * _NS
_RAYS_PER_W = _N_RAYS // _NW
_RAY_BLK = 16
_N_BLKS = _RAYS_PER_W // _RAY_BLK
_WORDS = _RES ** 3 // 32


def _pack_body(occ_ref, out_ref):
    x = occ_ref[...]
    j = lax.broadcasted_iota(jnp.int32, x.shape, 0)
    bits = jnp.where(x > _OCC_THRE, jnp.left_shift(jnp.int32(1), j),
                     jnp.int32(0))
    out_ref[...] = jnp.sum(bits, axis=0)


def _pack_bits(occs):
    occ3 = occs.reshape(32, 512, 128)
    words = pl.pallas_call(
        _pack_body,
        grid=(64,),
        in_specs=[pl.BlockSpec((32, 8, 128), lambda i: (0, i, 0))],
        out_specs=pl.BlockSpec((8, 128), lambda i: (i, 0)),
        out_shape=jax.ShapeDtypeStruct((512, 128), jnp.int32),
    )(occ3)
    return words.reshape(_WORDS)


def _march_body(o_h, d_h, words_h, out_h, bm, ov, dv, stage_a, stage_b,
                sem_a, sem_b):
    wid = lax.axis_index("s") * _NC + lax.axis_index("c")
    ray_base = wid * _RAYS_PER_W

    pltpu.sync_copy(words_h, bm)
    pltpu.sync_copy(o_h.at[pl.ds(ray_base * 3, _RAYS_PER_W * 3)], ov)
    pltpu.sync_copy(d_h.at[pl.ds(ray_base * 3, _RAYS_PER_W * 3)], dv)

    lane = lax.iota(jnp.int32, 16)
    zeros = jnp.zeros((16,), jnp.int32)
    lane256 = lane * (2 * _N_SAMPLES)

    def ray_block(rb, stage):
        off = pl.multiple_of(rb * _RAY_BLK, _RAY_BLK)
        idx3 = (off + lane) * 3
        ox = plsc.load_gather(ov, [idx3])
        oy = plsc.load_gather(ov, [idx3 + 1])
        oz = plsc.load_gather(ov, [idx3 + 2])
        dx = plsc.load_gather(dv, [idx3])
        dy = plsc.load_gather(dv, [idx3 + 1])
        dz = plsc.load_gather(dv, [idx3 + 2])

        def safe(d):
            return jnp.where(jnp.abs(d) < 1e-8, jnp.float32(1e-8), d)

        dxs, dys, dzs = safe(dx), safe(dy), safe(dz)
        t1x = (-1.0 - ox) / dxs
        t2x = (1.0 - ox) / dxs
        t1y = (-1.0 - oy) / dys
        t2y = (1.0 - oy) / dys
        t1z = (-1.0 - oz) / dzs
        t2z = (1.0 - oz) / dzs
        tmin = jnp.maximum(jnp.maximum(jnp.minimum(t1x, t2x),
                                       jnp.minimum(t1y, t2y)),
                           jnp.minimum(t1z, t2z))
        tmax = jnp.minimum(jnp.minimum(jnp.maximum(t1x, t2x),
                                       jnp.maximum(t1y, t2y)),
                           jnp.maximum(t1z, t2z))
        tmin = jnp.clip(tmin, 0.0, 1e10)
        tmax = jnp.clip(tmax, 0.0, 1e10)
        valid = tmax > tmin
        span = tmax - tmin

        @plsc.parallel_loop(0, _N_SAMPLES, 1, unroll=8)
        def sample(i):
            sf = i.astype(jnp.float32)
            t_s = tmin + (sf * (1.0 / _N_SAMPLES)) * span
            t_e = tmin + ((sf + 1.0) * (1.0 / _N_SAMPLES)) * span
            t_m = 0.5 * (t_s + t_e)
            px = ox + t_m * dx
            py = oy + t_m * dy
            pz = oz + t_m * dz
            ix = jnp.clip(((px + 1.0) * (0.5 * _RES)).astype(jnp.int32),
                          0, _RES - 1)
            iy = jnp.clip(((py + 1.0) * (0.5 * _RES)).astype(jnp.int32),
                          0, _RES - 1)
            iz = jnp.clip(((pz + 1.0) * (0.5 * _RES)).astype(jnp.int32),
                          0, _RES - 1)
            flat = (ix * _RES + iy) * _RES + iz
            w = jnp.bitwise_and(flat, _WORDS - 1)
            j = lax.shift_right_logical(flat, 16)
            word = plsc.load_gather(bm, [w])
            bit = jnp.bitwise_and(lax.shift_right_logical(word, j), 1)
            m = (bit != 0) & valid
            ts_o = jnp.where(m, t_s, 0.0)
            te_o = jnp.where(m, t_e, 0.0)
            idx = lane256 + i
            plsc.store_scatter(stage, [idx], ts_o)
            plsc.store_scatter(stage, [idx + _N_SAMPLES], te_o)

        return 0

    stage_n = _RAY_BLK * 2 * _N_SAMPLES

    def out_slice(rb):
        flat_base = (ray_base + rb * _RAY_BLK) * (2 * _N_SAMPLES)
        return out_h.at[pl.ds(flat_base, stage_n)]

    def two_blocks(k, _):
        rb_a = k * 2
        rb_b = rb_a + 1

        @pl.when(k > 0)
        def _():
            pltpu.make_async_copy(stage_a, out_slice(rb_a), sem_a).wait()

        ray_block(rb_a, stage_a)
        pltpu.async_copy(stage_a, out_slice(rb_a), sem_a)

        @pl.when(k > 0)
        def _():
            pltpu.make_async_copy(stage_b, out_slice(rb_b), sem_b).wait()

        ray_block(rb_b, stage_b)
        pltpu.async_copy(stage_b, out_slice(rb_b), sem_b)
        return 0

    lax.fori_loop(0, _N_BLKS // 2, two_blocks, 0)
    pltpu.make_async_copy(stage_a, out_slice(_N_BLKS - 2), sem_a).wait()
    pltpu.make_async_copy(stage_b, out_slice(_N_BLKS - 1), sem_b).wait()


@jax.jit
def kernel(rays_o, rays_d, occs):
    words = _pack_bits(occs)
    march = pl.kernel(
        _march_body,
        out_type=jax.ShapeDtypeStruct((_N_RAYS * _N_SAMPLES * 2,),
                                      jnp.float32),
        mesh=plsc.VectorSubcoreMesh(core_axis_name="c",
                                    subcore_axis_name="s"),
        compiler_params=pltpu.CompilerParams(needs_layout_passes=False),
        scratch_types=[
            pltpu.VMEM((_WORDS,), jnp.int32),
            pltpu.VMEM((_RAYS_PER_W * 3,), jnp.float32),
            pltpu.VMEM((_RAYS_PER_W * 3,), jnp.float32),
            pltpu.VMEM((_RAY_BLK * _N_SAMPLES * 2,), jnp.float32),
            pltpu.VMEM((_RAY_BLK * _N_SAMPLES * 2,), jnp.float32),
            pltpu.SemaphoreType.DMA,
            pltpu.SemaphoreType.DMA,
        ],
    )
    flat = march(rays_o.reshape(-1), rays_d.reshape(-1), words)
    return flat.reshape(_N_RAYS, 2, _N_SAMPLES).swapaxes(1, 2)


# speedup vs baseline: 1.1255x; 1.1255x over previous
import functools

import jax
import jax.numpy as jnp
from jax import lax
from jax.experimental import pallas as pl
from jax.experimental.pallas import tpu as pltpu
from jax.experimental.pallas import tpu_sc as plsc

_RES = 128
_N_RAYS = 65536
_N_SAMPLES = 128
_OCC_THRE = 0.5

_NC = 2
_NS = 16
_NW = _NC * _NS
_RAYS_PER_W = _N_RAYS // _NW
_RAY_BLK = 16
_N_BLKS = _RAYS_PER_W // _RAY_BLK
_WORDS = _RES ** 3 // 32


def _pack_body(occ_ref, out_ref):
    x = occ_ref[...]
    j = lax.broadcasted_iota(jnp.int32, x.shape, 0)
    bits = jnp.where(x > _OCC_THRE, jnp.left_shift(jnp.int32(1), j),
                     jnp.int32(0))
    out_ref[...] = jnp.sum(bits, axis=0)


def _pack_bits(occs):
    occ3 = occs.reshape(32, 512, 128)
    words = pl.pallas_call(
        _pack_body,
        grid=(64,),
        in_specs=[pl.BlockSpec((32, 8, 128), lambda i: (0, i, 0))],
        out_specs=pl.BlockSpec((8, 128), lambda i: (i, 0)),
        out_shape=jax.ShapeDtypeStruct((512, 128), jnp.int32),
    )(occ3)
    return words.reshape(_WORDS)


def _march_body(o_h, d_h, words_h, out_h, bm, ov, dv, stage_a, stage_b,
                sem_a, sem_b):
    wid = lax.axis_index("s") * _NC + lax.axis_index("c")
    ray_base = wid * _RAYS_PER_W

    pltpu.sync_copy(words_h, bm)
    pltpu.sync_copy(o_h.at[pl.ds(ray_base * 3, _RAYS_PER_W * 3)], ov)
    pltpu.sync_copy(d_h.at[pl.ds(ray_base * 3, _RAYS_PER_W * 3)], dv)

    lane = lax.iota(jnp.int32, 16)
    zeros = jnp.zeros((16,), jnp.int32)
    lane256 = lane * (2 * _N_SAMPLES)

    def ray_block(rb, stage):
        off = pl.multiple_of(rb * _RAY_BLK, _RAY_BLK)
        idx3 = (off + lane) * 3
        ox = plsc.load_gather(ov, [idx3])
        oy = plsc.load_gather(ov, [idx3 + 1])
        oz = plsc.load_gather(ov, [idx3 + 2])
        dx = plsc.load_gather(dv, [idx3])
        dy = plsc.load_gather(dv, [idx3 + 1])
        dz = plsc.load_gather(dv, [idx3 + 2])

        def safe(d):
            return jnp.where(jnp.abs(d) < 1e-8, jnp.float32(1e-8), d)

        dxs, dys, dzs = safe(dx), safe(dy), safe(dz)
        t1x = (-1.0 - ox) / dxs
        t2x = (1.0 - ox) / dxs
        t1y = (-1.0 - oy) / dys
        t2y = (1.0 - oy) / dys
        t1z = (-1.0 - oz) / dzs
        t2z = (1.0 - oz) / dzs
        tmin = jnp.maximum(jnp.maximum(jnp.minimum(t1x, t2x),
                                       jnp.minimum(t1y, t2y)),
                           jnp.minimum(t1z, t2z))
        tmax = jnp.minimum(jnp.minimum(jnp.maximum(t1x, t2x),
                                       jnp.maximum(t1y, t2y)),
                           jnp.maximum(t1z, t2z))
        tmin = jnp.clip(tmin, 0.0, 1e10)
        tmax = jnp.clip(tmax, 0.0, 1e10)
        valid = tmax > tmin
        span = tmax - tmin

        @plsc.parallel_loop(0, _N_SAMPLES, 1, unroll=4)
        def sample(i):
            sf = i.astype(jnp.float32)
            t_s = tmin + (sf * (1.0 / _N_SAMPLES)) * span
            t_e = tmin + ((sf + 1.0) * (1.0 / _N_SAMPLES)) * span
            t_m = 0.5 * (t_s + t_e)
            px = ox + t_m * dx
            py = oy + t_m * dy
            pz = oz + t_m * dz
            def cell(pf):
                i32 = ((pf + 1.0) * (0.5 * _RES)).astype(jnp.int32)
                u = plsc.bitcast(i32, jnp.uint32)
                return plsc.bitcast(jnp.minimum(u, jnp.uint32(_RES - 1)),
                                    jnp.int32)

            ix = cell(px)
            iy = cell(py)
            iz = cell(pz)
            flat = (ix * _RES + iy) * _RES + iz
            w = jnp.bitwise_and(flat, _WORDS - 1)
            j = lax.shift_right_logical(ix, 2)
            word = plsc.load_gather(bm, [w])
            bit = jnp.bitwise_and(lax.shift_right_logical(word, j), 1)
            m = (bit != 0) & valid
            ts_o = jnp.where(m, t_s, 0.0)
            te_o = jnp.where(m, t_e, 0.0)
            idx = lane256 + i
            plsc.store_scatter(stage, [idx], ts_o)
            plsc.store_scatter(stage, [idx + _N_SAMPLES], te_o)

        return 0

    stage_n = _RAY_BLK * 2 * _N_SAMPLES

    def out_slice(rb):
        flat_base = (ray_base + rb * _RAY_BLK) * (2 * _N_SAMPLES)
        return out_h.at[pl.ds(flat_base, stage_n)]

    def two_blocks(k, _):
        rb_a = k * 2
        rb_b = rb_a + 1

        @pl.when(k > 0)
        def _():
            pltpu.make_async_copy(stage_a, out_slice(rb_a), sem_a).wait()

        ray_block(rb_a, stage_a)
        pltpu.async_copy(stage_a, out_slice(rb_a), sem_a)

        @pl.when(k > 0)
        def _():
            pltpu.make_async_copy(stage_b, out_slice(rb_b), sem_b).wait()

        ray_block(rb_b, stage_b)
        pltpu.async_copy(stage_b, out_slice(rb_b), sem_b)
        return 0

    lax.fori_loop(0, _N_BLKS // 2, two_blocks, 0)
    pltpu.make_async_copy(stage_a, out_slice(_N_BLKS - 2), sem_a).wait()
    pltpu.make_async_copy(stage_b, out_slice(_N_BLKS - 1), sem_b).wait()


@jax.jit
def kernel(rays_o, rays_d, occs):
    words = _pack_bits(occs)
    march = pl.kernel(
        _march_body,
        out_type=jax.ShapeDtypeStruct((_N_RAYS * _N_SAMPLES * 2,),
                                      jnp.float32),
        mesh=plsc.VectorSubcoreMesh(core_axis_name="c",
                                    subcore_axis_name="s"),
        compiler_params=pltpu.CompilerParams(needs_layout_passes=False),
        scratch_types=[
            pltpu.VMEM((_WORDS,), jnp.int32),
            pltpu.VMEM((_RAYS_PER_W * 3,), jnp.float32),
            pltpu.VMEM((_RAYS_PER_W * 3,), jnp.float32),
            pltpu.VMEM((_RAY_BLK * _N_SAMPLES * 2,), jnp.float32),
            pltpu.VMEM((_RAY_BLK * _N_SAMPLES * 2,), jnp.float32),
            pltpu.SemaphoreType.DMA,
            pltpu.SemaphoreType.DMA,
        ],
    )
    flat = march(rays_o.reshape(-1), rays_d.reshape(-1), words)
    return flat.reshape(_N_RAYS, 2, _N_SAMPLES).swapaxes(1, 2)


# speedup vs baseline: 1.1596x; 1.0303x over previous
import functools

import jax
import jax.numpy as jnp
from jax import lax
from jax.experimental import pallas as pl
from jax.experimental.pallas import tpu as pltpu
from jax.experimental.pallas import tpu_sc as plsc

_RES = 128
_N_RAYS = 65536
_N_SAMPLES = 128
_OCC_THRE = 0.5

_NC = 2
_NS = 16
_NW = _NC * _NS
_RAYS_PER_W = _N_RAYS // _NW
_RAY_BLK = 16
_N_BLKS = _RAYS_PER_W // _RAY_BLK
_WORDS = _RES ** 3 // 32


def _pack_body(occ_ref, out_ref):
    x = occ_ref[...]
    j = lax.broadcasted_iota(jnp.int32, x.shape, 0)
    bits = jnp.where(x > _OCC_THRE, jnp.left_shift(jnp.int32(1), j),
                     jnp.int32(0))
    out_ref[...] = jnp.sum(bits, axis=0)


def _pack_bits(occs):
    occ3 = occs.reshape(32, 512, 128)
    words = pl.pallas_call(
        _pack_body,
        grid=(64,),
        in_specs=[pl.BlockSpec((32, 8, 128), lambda i: (0, i, 0))],
        out_specs=pl.BlockSpec((8, 128), lambda i: (i, 0)),
        out_shape=jax.ShapeDtypeStruct((512, 128), jnp.int32),
    )(occ3)
    return words.reshape(_WORDS)


def _march_body(o_h, d_h, words_h, out_h, bm, ov, dv, stage_a, stage_b,
                sem_a, sem_b):
    wid = lax.axis_index("s") * _NC + lax.axis_index("c")
    ray_base = wid * _RAYS_PER_W

    pltpu.sync_copy(words_h, bm)
    pltpu.sync_copy(o_h.at[pl.ds(ray_base * 3, _RAYS_PER_W * 3)], ov)
    pltpu.sync_copy(d_h.at[pl.ds(ray_base * 3, _RAYS_PER_W * 3)], dv)

    lane = lax.iota(jnp.int32, 16)
    zeros = jnp.zeros((16,), jnp.int32)
    lane256 = lane * (2 * _N_SAMPLES)

    def ray_block(rb, stage):
        off = pl.multiple_of(rb * _RAY_BLK, _RAY_BLK)
        idx3 = (off + lane) * 3
        ox = plsc.load_gather(ov, [idx3])
        oy = plsc.load_gather(ov, [idx3 + 1])
        oz = plsc.load_gather(ov, [idx3 + 2])
        dx = plsc.load_gather(dv, [idx3])
        dy = plsc.load_gather(dv, [idx3 + 1])
        dz = plsc.load_gather(dv, [idx3 + 2])

        def safe(d):
            return jnp.where(jnp.abs(d) < 1e-8, jnp.float32(1e-8), d)

        dxs, dys, dzs = safe(dx), safe(dy), safe(dz)
        t1x = (-1.0 - ox) / dxs
        t2x = (1.0 - ox) / dxs
        t1y = (-1.0 - oy) / dys
        t2y = (1.0 - oy) / dys
        t1z = (-1.0 - oz) / dzs
        t2z = (1.0 - oz) / dzs
        tmin = jnp.maximum(jnp.maximum(jnp.minimum(t1x, t2x),
                                       jnp.minimum(t1y, t2y)),
                           jnp.minimum(t1z, t2z))
        tmax = jnp.minimum(jnp.minimum(jnp.maximum(t1x, t2x),
                                       jnp.maximum(t1y, t2y)),
                           jnp.maximum(t1z, t2z))
        tmin = jnp.clip(tmin, 0.0, 1e10)
        tmax = jnp.clip(tmax, 0.0, 1e10)
        valid = tmax > tmin
        span = tmax - tmin

        @plsc.parallel_loop(0, _N_SAMPLES, 1, unroll=4)
        def sample(i):
            sf = i.astype(jnp.float32)
            t_s = tmin + (sf * (1.0 / _N_SAMPLES)) * span
            t_e = tmin + ((sf + 1.0) * (1.0 / _N_SAMPLES)) * span
            t_m = 0.5 * (t_s + t_e)
            px = ox + t_m * dx
            py = oy + t_m * dy
            pz = oz + t_m * dz
            ix = jnp.clip(((px + 1.0) * (0.5 * _RES)).astype(jnp.int32),
                          0, _RES - 1)
            iy = jnp.clip(((py + 1.0) * (0.5 * _RES)).astype(jnp.int32),
                          0, _RES - 1)
            iz = jnp.clip(((pz + 1.0) * (0.5 * _RES)).astype(jnp.int32),
                          0, _RES - 1)
            flat = (ix * _RES + iy) * _RES + iz
            w = jnp.bitwise_and(flat, _WORDS - 1)
            j = lax.shift_right_logical(ix, 2)
            word = plsc.load_gather(bm, [w])
            bit = jnp.bitwise_and(lax.shift_right_logical(word, j), 1)
            m = (bit != 0) & valid
            ts_o = jnp.where(m, t_s, 0.0)
            te_o = jnp.where(m, t_e, 0.0)
            idx = lane256 + i
            plsc.store_scatter(stage, [idx], ts_o)
            plsc.store_scatter(stage, [idx + _N_SAMPLES], te_o)

        return 0

    stage_n = _RAY_BLK * 2 * _N_SAMPLES

    def out_slice(rb):
        flat_base = (ray_base + rb * _RAY_BLK) * (2 * _N_SAMPLES)
        return out_h.at[pl.ds(flat_base, stage_n)]

    def two_blocks(k, _):
        rb_a = k * 2
        rb_b = rb_a + 1

        @pl.when(k > 0)
        def _():
            pltpu.make_async_copy(stage_a, out_slice(rb_a), sem_a).wait()

        ray_block(rb_a, stage_a)
        pltpu.async_copy(stage_a, out_slice(rb_a), sem_a)

        @pl.when(k > 0)
        def _():
            pltpu.make_async_copy(stage_b, out_slice(rb_b), sem_b).wait()

        ray_block(rb_b, stage_b)
        pltpu.async_copy(stage_b, out_slice(rb_b), sem_b)
        return 0

    lax.fori_loop(0, _N_BLKS // 2, two_blocks, 0)
    pltpu.make_async_copy(stage_a, out_slice(_N_BLKS - 2), sem_a).wait()
    pltpu.make_async_copy(stage_b, out_slice(_N_BLKS - 1), sem_b).wait()


@jax.jit
def kernel(rays_o, rays_d, occs):
    words = _pack_bits(occs)
    march = pl.kernel(
        _march_body,
        out_type=jax.ShapeDtypeStruct((_N_RAYS * _N_SAMPLES * 2,),
                                      jnp.float32),
        mesh=plsc.VectorSubcoreMesh(core_axis_name="c",
                                    subcore_axis_name="s"),
        compiler_params=pltpu.CompilerParams(needs_layout_passes=False),
        scratch_types=[
            pltpu.VMEM((_WORDS,), jnp.int32),
            pltpu.VMEM((_RAYS_PER_W * 3,), jnp.float32),
            pltpu.VMEM((_RAYS_PER_W * 3,), jnp.float32),
            pltpu.VMEM((_RAY_BLK * _N_SAMPLES * 2,), jnp.float32),
            pltpu.VMEM((_RAY_BLK * _N_SAMPLES * 2,), jnp.float32),
            pltpu.SemaphoreType.DMA,
            pltpu.SemaphoreType.DMA,
        ],
    )
    flat = march(rays_o.reshape(-1), rays_d.reshape(-1), words)
    return flat.reshape(_N_RAYS, 2, _N_SAMPLES).swapaxes(1, 2)


# speedup vs baseline: 1.4335x; 1.2362x over previous
import functools

import jax
import jax.numpy as jnp
from jax import lax
from jax.experimental import pallas as pl
from jax.experimental.pallas import tpu as pltpu
from jax.experimental.pallas import tpu_sc as plsc

_RES = 128
_N_RAYS = 65536
_N_SAMPLES = 128
_OCC_THRE = 0.5

_NC = 2
_NS = 16
_NW = _NC * _NS
_RAYS_PER_W = _N_RAYS // _NW
_RAY_BLK = 16
_N_BLKS = _RAYS_PER_W // _RAY_BLK
_WORDS = _RES ** 3 // 32


def _pack_body(occ_ref, out_ref):
    x = occ_ref[...]
    j = lax.broadcasted_iota(jnp.int32, x.shape, 0)
    bits = jnp.where(x > _OCC_THRE, jnp.left_shift(jnp.int32(1), j),
                     jnp.int32(0))
    out_ref[...] = jnp.sum(bits, axis=0)


def _pack_bits(occs):
    occ3 = occs.reshape(32, 512, 128)
    words = pl.pallas_call(
        _pack_body,
        grid=(64,),
        in_specs=[pl.BlockSpec((32, 8, 128), lambda i: (0, i, 0))],
        out_specs=pl.BlockSpec((8, 128), lambda i: (i, 0)),
        out_shape=jax.ShapeDtypeStruct((512, 128), jnp.int32),
    )(occ3)
    return words.reshape(_WORDS)


def _march_body(o_h, d_h, words_h, out_h, bm, ov, dv, stage_a, stage_b,
                sem_a, sem_b):
    wid = lax.axis_index("s") * _NC + lax.axis_index("c")
    ray_base = wid * _RAYS_PER_W

    pltpu.sync_copy(words_h, bm)
    pltpu.sync_copy(o_h.at[:, pl.ds(ray_base, _RAYS_PER_W)], ov)
    pltpu.sync_copy(d_h.at[:, pl.ds(ray_base, _RAYS_PER_W)], dv)

    lane = lax.iota(jnp.int32, 16)
    zeros = jnp.zeros((16,), jnp.int32)
    lane256 = lane * (2 * _N_SAMPLES)

    def ray_block(rb, stage):
        off = pl.multiple_of(rb * _RAY_BLK, _RAY_BLK)
        ox = ov[0, pl.ds(off, 16)]
        oy = ov[1, pl.ds(off, 16)]
        oz = ov[2, pl.ds(off, 16)]
        dx = dv[0, pl.ds(off, 16)]
        dy = dv[1, pl.ds(off, 16)]
        dz = dv[2, pl.ds(off, 16)]

        def safe(d):
            return jnp.where(jnp.abs(d) < 1e-8, jnp.float32(1e-8), d)

        dxs, dys, dzs = safe(dx), safe(dy), safe(dz)
        t1x = (-1.0 - ox) / dxs
        t2x = (1.0 - ox) / dxs
        t1y = (-1.0 - oy) / dys
        t2y = (1.0 - oy) / dys
        t1z = (-1.0 - oz) / dzs
        t2z = (1.0 - oz) / dzs
        tmin = jnp.maximum(jnp.maximum(jnp.minimum(t1x, t2x),
                                       jnp.minimum(t1y, t2y)),
                           jnp.minimum(t1z, t2z))
        tmax = jnp.minimum(jnp.minimum(jnp.maximum(t1x, t2x),
                                       jnp.maximum(t1y, t2y)),
                           jnp.maximum(t1z, t2z))
        tmin = jnp.clip(tmin, 0.0, 1e10)
        tmax = jnp.clip(tmax, 0.0, 1e10)
        valid = tmax > tmin
        span = tmax - tmin

        @plsc.parallel_loop(0, _N_SAMPLES, 1, unroll=4)
        def sample(i):
            sf = i.astype(jnp.float32)
            t_s = tmin + (sf * (1.0 / _N_SAMPLES)) * span
            t_e = tmin + ((sf + 1.0) * (1.0 / _N_SAMPLES)) * span
            t_m = 0.5 * (t_s + t_e)
            px = ox + t_m * dx
            py = oy + t_m * dy
            pz = oz + t_m * dz
            ix = jnp.clip(((px + 1.0) * (0.5 * _RES)).astype(jnp.int32),
                          0, _RES - 1)
            iy = jnp.clip(((py + 1.0) * (0.5 * _RES)).astype(jnp.int32),
                          0, _RES - 1)
            iz = jnp.clip(((pz + 1.0) * (0.5 * _RES)).astype(jnp.int32),
                          0, _RES - 1)
            flat = (ix * _RES + iy) * _RES + iz
            w = jnp.bitwise_and(flat, _WORDS - 1)
            j = lax.shift_right_logical(ix, 2)
            word = plsc.load_gather(bm, [w])
            bit = jnp.bitwise_and(lax.shift_right_logical(word, j), 1)
            m = (bit != 0) & valid
            ts_o = jnp.where(m, t_s, 0.0)
            te_o = jnp.where(m, t_e, 0.0)
            idx = lane256 + i
            plsc.store_scatter(stage, [idx], ts_o)
            plsc.store_scatter(stage, [idx + _N_SAMPLES], te_o)

        return 0

    stage_n = _RAY_BLK * 2 * _N_SAMPLES

    def out_slice(rb):
        flat_base = (ray_base + rb * _RAY_BLK) * (2 * _N_SAMPLES)
        return out_h.at[pl.ds(flat_base, stage_n)]

    def two_blocks(k, _):
        rb_a = k * 2
        rb_b = rb_a + 1

        @pl.when(k > 0)
        def _():
            pltpu.make_async_copy(stage_a, out_slice(rb_a), sem_a).wait()

        ray_block(rb_a, stage_a)
        pltpu.async_copy(stage_a, out_slice(rb_a), sem_a)

        @pl.when(k > 0)
        def _():
            pltpu.make_async_copy(stage_b, out_slice(rb_b), sem_b).wait()

        ray_block(rb_b, stage_b)
        pltpu.async_copy(stage_b, out_slice(rb_b), sem_b)
        return 0

    lax.fori_loop(0, _N_BLKS // 2, two_blocks, 0)
    pltpu.make_async_copy(stage_a, out_slice(_N_BLKS - 2), sem_a).wait()
    pltpu.make_async_copy(stage_b, out_slice(_N_BLKS - 1), sem_b).wait()


@jax.jit
def kernel(rays_o, rays_d, occs):
    words = _pack_bits(occs)
    march = pl.kernel(
        _march_body,
        out_type=jax.ShapeDtypeStruct((_N_RAYS * _N_SAMPLES * 2,),
                                      jnp.float32),
        mesh=plsc.VectorSubcoreMesh(core_axis_name="c",
                                    subcore_axis_name="s"),
        compiler_params=pltpu.CompilerParams(needs_layout_passes=False),
        scratch_types=[
            pltpu.VMEM((_WORDS,), jnp.int32),
            pltpu.VMEM((3, _RAYS_PER_W), jnp.float32),
            pltpu.VMEM((3, _RAYS_PER_W), jnp.float32),
            pltpu.VMEM((_RAY_BLK * _N_SAMPLES * 2,), jnp.float32),
            pltpu.VMEM((_RAY_BLK * _N_SAMPLES * 2,), jnp.float32),
            pltpu.SemaphoreType.DMA,
            pltpu.SemaphoreType.DMA,
        ],
    )
    flat = march(rays_o.T, rays_d.T, words)
    return flat.reshape(_N_RAYS, 2, _N_SAMPLES).swapaxes(1, 2)
